# baseline (speedup 1.0000x reference)
import functools
import math

import jax
import jax.numpy as jnp
import numpy as np
from jax import lax
from jax.experimental import pallas as pl
from jax.experimental.pallas import tpu as pltpu
from jax.experimental.pallas import tpu_sc as plsc

DMODEL = 64
MAX_SEQ_LEN = 200
ZERO_ROW = MAX_SEQ_LEN
PE_ROWS = MAX_SEQ_LEN + 8

NUM_WORKERS = 32
CHUNK = 128


@functools.lru_cache(maxsize=None)
def _pe_table():
    position = np.arange(MAX_SEQ_LEN, dtype=np.float64)[:, None]
    div_term = np.exp(
        np.arange(0, DMODEL, 2, dtype=np.float64) * (-math.log(10000.0) / DMODEL)
    )
    pe = np.zeros((PE_ROWS, DMODEL), dtype=np.float32)
    pe[:MAX_SEQ_LEN, 0::2] = np.sin(position * div_term)
    pe[:MAX_SEQ_LEN, 1::2] = np.cos(position * div_term)
    return jnp.asarray(pe)


@functools.lru_cache(maxsize=None)
def _build_gather(n_rows):
    assert n_rows % (NUM_WORKERS * CHUNK) == 0
    rows_per_worker = n_rows // NUM_WORKERS
    n_chunks = rows_per_worker // CHUNK
    mesh = plsc.VectorSubcoreMesh(core_axis_name="c", subcore_axis_name="s")

    def body(emb_hbm, pe_hbm, idx_hbm, p1_hbm, p2_hbm, out_hbm,
             pe_t, idx_v, p1i_v, p2i_v, rows_v, sem):
        wid = lax.axis_index("s") * 2 + lax.axis_index("c")
        w_base = wid * rows_per_worker
        pltpu.sync_copy(pe_hbm, pe_t)

        def chunk(i, carry):
            base = w_base + i * CHUNK
            pltpu.sync_copy(idx_hbm.at[pl.ds(base, CHUNK)], idx_v)
            pltpu.sync_copy(p1_hbm.at[pl.ds(base, CHUNK)], p1i_v)
            pltpu.sync_copy(p2_hbm.at[pl.ds(base, CHUNK)], p2i_v)
            cp0 = pltpu.async_copy(emb_hbm.at[idx_v], rows_v, sem)
            cp0.wait()

            @plsc.parallel_loop(0, CHUNK // 16, unroll=2)
            def add_group(g):
                p1vec = p1i_v[pl.ds(g * 16, 16)]
                p2vec = p2i_v[pl.ds(g * 16, 16)]
                for j in range(16):
                    r = g * 16 + j
                    p1r = p1vec[j]
                    p2r = p2vec[j]
                    for c in range(DMODEL // 16):
                        sl = pl.ds(c * 16, 16)
                        rows_v[r, sl] = rows_v[r, sl] + pe_t[p1r, sl] + pe_t[p2r, sl]

            pltpu.sync_copy(rows_v, out_hbm.at[pl.ds(base, CHUNK)])
            return carry

        lax.fori_loop(0, n_chunks, chunk, 0)

    return pl.kernel(
        body,
        out_type=jax.ShapeDtypeStruct((n_rows, DMODEL), jnp.float32),
        mesh=mesh,
        compiler_params=pltpu.CompilerParams(use_tc_tiling_on_sc=False),
        scratch_types=[
            pltpu.VMEM((PE_ROWS, DMODEL), jnp.float32),
            pltpu.VMEM((CHUNK,), jnp.int32),
            pltpu.VMEM((CHUNK,), jnp.int32),
            pltpu.VMEM((CHUNK,), jnp.int32),
            pltpu.VMEM((CHUNK, DMODEL), jnp.float32),
            pltpu.SemaphoreType.DMA,
        ],
    )


def kernel(nodes, edges, children, parents, emb_table):
    bsz, seq_len = nodes.shape
    n_edges = edges.shape[1]
    tokens = seq_len + n_edges
    n_rows = bsz * tokens
    pe = _pe_table()

    node_pos = jnp.broadcast_to(
        jnp.arange(seq_len, dtype=jnp.int32)[None, :], (bsz, seq_len)
    )
    idx = jnp.concatenate(
        [nodes.astype(jnp.int32), edges.astype(jnp.int32)], axis=1
    ).reshape(-1)
    p1 = jnp.concatenate([node_pos, children.astype(jnp.int32)], axis=1).reshape(-1)
    p2 = jnp.concatenate(
        [jnp.full((bsz, seq_len), ZERO_ROW, jnp.int32), parents.astype(jnp.int32)],
        axis=1,
    ).reshape(-1)

    out = _build_gather(n_rows)(emb_table, pe, idx, p1, p2)
    return out.reshape(bsz, tokens, DMODEL)


# speedup vs baseline: 1.3649x; 1.3649x over previous
import functools
import math

import jax
import jax.numpy as jnp
import numpy as np
from jax import lax
from jax.experimental import pallas as pl
from jax.experimental.pallas import tpu as pltpu
from jax.experimental.pallas import tpu_sc as plsc

DMODEL = 64
MAX_SEQ_LEN = 200
ZERO_ROW = MAX_SEQ_LEN
PE_ROWS = MAX_SEQ_LEN + 8

NUM_WORKERS = 32
CHUNK = 128
NBUF = 3


@functools.lru_cache(maxsize=None)
def _pe_table():
    position = np.arange(MAX_SEQ_LEN, dtype=np.float64)[:, None]
    div_term = np.exp(
        np.arange(0, DMODEL, 2, dtype=np.float64) * (-math.log(10000.0) / DMODEL)
    )
    pe = np.zeros((PE_ROWS, DMODEL), dtype=np.float32)
    pe[:MAX_SEQ_LEN, 0::2] = np.sin(position * div_term)
    pe[:MAX_SEQ_LEN, 1::2] = np.cos(position * div_term)
    return jnp.asarray(pe)


@functools.lru_cache(maxsize=None)
def _build_gather(n_rows):
    assert n_rows % (NUM_WORKERS * CHUNK) == 0
    rows_per_worker = n_rows // NUM_WORKERS
    n_chunks = rows_per_worker // CHUNK
    assert n_chunks % NBUF == 0
    n_groups = n_chunks // NBUF
    mesh = plsc.VectorSubcoreMesh(core_axis_name="c", subcore_axis_name="s")

    def body(emb_hbm, pe_hbm, combo_hbm, out_hbm,
             pe_t, combo_v, rows_v, semi, semg, sems):
        wid = lax.axis_index("s") * 2 + lax.axis_index("c")
        w_base = wid * rows_per_worker
        w_chunk0 = wid * n_chunks
        max_chunk = w_chunk0 + n_chunks - 1
        pltpu.sync_copy(pe_hbm, pe_t)

        def fire_idx(c, b):
            cc = lax.min(w_chunk0 + c, max_chunk)
            pltpu.async_copy(combo_hbm.at[cc], combo_v.at[b], semi.at[b])

        def wait_idx(b):
            pltpu.make_async_copy(combo_hbm.at[0], combo_v.at[b], semi.at[b]).wait()

        for b in range(NBUF):
            fire_idx(b, b)

        def group(g, carry):
            c0 = g * NBUF
            gathers = []
            for b in range(NBUF):
                wait_idx(b)
                gathers.append(
                    pltpu.async_copy(emb_hbm.at[combo_v.at[b, 0]], rows_v.at[b],
                                     semg.at[b])
                )
            stores = []
            for b in range(NBUF):
                gathers[b].wait()

                @plsc.parallel_loop(0, CHUNK // 16, unroll=2)
                def add_group(gg, _b=b):
                    p1vec = combo_v[_b, 1, pl.ds(gg * 16, 16)]
                    p2vec = combo_v[_b, 2, pl.ds(gg * 16, 16)]
                    for j in range(16):
                        r = gg * 16 + j
                        p1r = p1vec[j]
                        p2r = p2vec[j]
                        for c in range(DMODEL // 16):
                            sl = pl.ds(c * 16, 16)
                            rows_v[_b, r, sl] = (
                                rows_v[_b, r, sl] + pe_t[p1r, sl] + pe_t[p2r, sl]
                            )

                base = w_base + (c0 + b) * CHUNK
                stores.append(
                    pltpu.async_copy(rows_v.at[b], out_hbm.at[pl.ds(base, CHUNK)],
                                     sems.at[b])
                )
            for b in range(NBUF):
                fire_idx(c0 + NBUF + b, b)
            for b in range(NBUF):
                stores[b].wait()
            return carry

        lax.fori_loop(0, n_groups, group, 0)
        for b in range(NBUF):
            wait_idx(b)

    return pl.kernel(
        body,
        out_type=jax.ShapeDtypeStruct((n_rows, DMODEL), jnp.float32),
        mesh=mesh,
        compiler_params=pltpu.CompilerParams(use_tc_tiling_on_sc=False),
        scratch_types=[
            pltpu.VMEM((PE_ROWS, DMODEL), jnp.float32),
            pltpu.VMEM((NBUF, 3, CHUNK), jnp.int32),
            pltpu.VMEM((NBUF, CHUNK, DMODEL), jnp.float32),
            pltpu.SemaphoreType.DMA((NBUF,)),
            pltpu.SemaphoreType.DMA((NBUF,)),
            pltpu.SemaphoreType.DMA((NBUF,)),
        ],
    )


def kernel(nodes, edges, children, parents, emb_table):
    bsz, seq_len = nodes.shape
    n_edges = edges.shape[1]
    tokens = seq_len + n_edges
    n_rows = bsz * tokens
    pe = _pe_table()

    node_pos = jnp.broadcast_to(
        jnp.arange(seq_len, dtype=jnp.int32)[None, :], (bsz, seq_len)
    )
    idx = jnp.concatenate(
        [nodes.astype(jnp.int32), edges.astype(jnp.int32)], axis=1
    ).reshape(-1, CHUNK)
    p1 = jnp.concatenate(
        [node_pos, children.astype(jnp.int32)], axis=1
    ).reshape(-1, CHUNK)
    p2 = jnp.concatenate(
        [jnp.full((bsz, seq_len), ZERO_ROW, jnp.int32), parents.astype(jnp.int32)],
        axis=1,
    ).reshape(-1, CHUNK)
    combo = jnp.stack([idx, p1, p2], axis=1)

    out = _build_gather(n_rows)(emb_table, pe, combo)
    return out.reshape(bsz, tokens, DMODEL)
